# baseline (speedup 1.0000x reference)
import functools

import jax
import jax.numpy as jnp
from jax import lax
from jax.experimental import pallas as pl
from jax.experimental.pallas import tpu as pltpu
from jax.experimental.pallas import tpu_sc as plsc

NUM_CODES = 8192
LATENT_DIM = 256
NUM_TOKENS = 8192
BETA = 0.25


_TOK_TILE = 2048
_CODE_TILE = 4096
_LANES = 128
_HIST_CHUNK = 1024


def _main_body(z_ref, w_ref, zsum2_ref, wsum2_ref,
               idx_ref, loss_ref, perp_ref,
               acc_val, acc_idx, counts, loss_sum):
    i = pl.program_id(0)
    j = pl.program_id(1)
    last_i = pl.num_programs(0) - 1
    last_j = pl.num_programs(1) - 1

    @pl.when(jnp.logical_and(i == 0, j == 0))
    def _():
        counts[...] = jnp.zeros(counts.shape, jnp.float32)
        loss_sum[0, 0] = 0.0

    @pl.when(j == 0)
    def _():
        acc_val[...] = jnp.full(acc_val.shape, jnp.inf, jnp.float32)
        acc_idx[...] = jnp.zeros(acc_idx.shape, jnp.int32)

    dot = lax.dot_general(z_ref[...], w_ref[...],
                          (((1,), (1,)), ((), ())),
                          preferred_element_type=jnp.float32)

    zs2 = zsum2_ref[...]
    av = acc_val[...]
    ai = acc_idx[...]
    for c in range(_CODE_TILE // _LANES):
        sl = slice(c * _LANES, (c + 1) * _LANES)
        d2 = (zs2 + wsum2_ref[:, sl]) - dot[:, sl]
        ioc = (lax.broadcasted_iota(jnp.int32, d2.shape, 1)
               + j * _CODE_TILE + c * _LANES)
        upd = d2 < av
        ai = jnp.where(upd, ioc, ai)
        av = jnp.where(upd, d2, av)

    @pl.when(j != last_j)
    def _():
        acc_val[...] = av
        acc_idx[...] = ai

    @pl.when(j == last_j)
    def _():
        m = jnp.min(av, axis=1, keepdims=True)
        idxt = jnp.min(jnp.where(av == m, ai, jnp.int32(2**30)),
                       axis=1, keepdims=True)
        idx_ref[...] = idxt
        loss_sum[0, 0] += jnp.sum(m) * 2.0
        for c0 in range(0, NUM_CODES, _HIST_CHUNK):
            codes = lax.broadcasted_iota(
                jnp.int32, (1, _HIST_CHUNK), 1) + c0
            eq = idxt == codes
            counts[:, c0:c0 + _HIST_CHUNK] += jnp.sum(
                jnp.where(eq, 1.0, 0.0), axis=0, keepdims=True)

    @pl.when(jnp.logical_and(i == last_i, j == last_j))
    def _():
        p = counts[...] * (1.0 / NUM_TOKENS)
        ent = jnp.sum(p * jnp.log(p + 1e-10))
        perp_ref[0, 0] = jnp.exp(-ent)
        mean = loss_sum[0, 0] * (1.0 / (NUM_TOKENS * LATENT_DIM))
        loss_ref[0, 0] = mean + BETA * mean


def _dist_argmin_stats(z_flat, w, zsum2, wsum2):
    grid = (NUM_TOKENS // _TOK_TILE, NUM_CODES // _CODE_TILE)
    return pl.pallas_call(
        _main_body,
        grid=grid,
        in_specs=[
            pl.BlockSpec((_TOK_TILE, LATENT_DIM), lambda i, j: (i, 0)),
            pl.BlockSpec((_CODE_TILE, LATENT_DIM), lambda i, j: (j, 0)),
            pl.BlockSpec((_TOK_TILE, 1), lambda i, j: (i, 0)),
            pl.BlockSpec((1, _CODE_TILE), lambda i, j: (0, j)),
        ],
        out_specs=[
            pl.BlockSpec((_TOK_TILE, 1), lambda i, j: (i, 0)),
            pl.BlockSpec(memory_space=pltpu.SMEM),
            pl.BlockSpec(memory_space=pltpu.SMEM),
        ],
        out_shape=[
            jax.ShapeDtypeStruct((NUM_TOKENS, 1), jnp.int32),
            jax.ShapeDtypeStruct((1, 1), jnp.float32),
            jax.ShapeDtypeStruct((1, 1), jnp.float32),
        ],
        scratch_shapes=[
            pltpu.VMEM((_TOK_TILE, _LANES), jnp.float32),
            pltpu.VMEM((_TOK_TILE, _LANES), jnp.int32),
            pltpu.VMEM((1, NUM_CODES), jnp.float32),
            pltpu.SMEM((1, 1), jnp.float32),
        ],
        compiler_params=pltpu.CompilerParams(
            dimension_semantics=("arbitrary", "arbitrary")),
    )(z_flat, w, zsum2, wsum2)


_SC_NUM_CORES = 2
_SC_NUM_SUBCORES = 16


def _sc_gather(table, idx):
    nw = _SC_NUM_CORES * _SC_NUM_SUBCORES
    b_per_w = NUM_TOKENS // nw
    mesh = plsc.VectorSubcoreMesh(core_axis_name="c", subcore_axis_name="s",
                                  num_cores=_SC_NUM_CORES,
                                  num_subcores=_SC_NUM_SUBCORES)

    @functools.partial(
        pl.kernel,
        mesh=mesh,
        out_type=jax.ShapeDtypeStruct((NUM_TOKENS, LATENT_DIM), jnp.float32),
        scratch_types=[
            pltpu.VMEM((b_per_w,), jnp.int32),
            pltpu.VMEM((b_per_w, LATENT_DIM), jnp.float32),
            pltpu.SemaphoreType.DMA,
        ],
    )
    def gather_kernel(table_hbm, idx_hbm, out_hbm, idx_v, rows_v, sem):
        wid = lax.axis_index("s") * _SC_NUM_CORES + lax.axis_index("c")
        base = wid * b_per_w
        pltpu.sync_copy(idx_hbm.at[pl.ds(base, b_per_w)], idx_v)
        pltpu.async_copy(table_hbm.at[idx_v], rows_v, sem).wait()
        pltpu.sync_copy(rows_v, out_hbm.at[pl.ds(base, b_per_w)])

    return gather_kernel(table, idx)


def kernel(z, embedding_weight):
    norm = jnp.sqrt(jnp.sum(embedding_weight ** 2, axis=1, keepdims=True))
    w = embedding_weight / jnp.maximum(norm, 1e-12)

    z_perm = jnp.transpose(z, (0, 2, 3, 1))
    z_flat = z_perm.reshape(-1, LATENT_DIM)

    zsum2 = jnp.sum(z_flat ** 2, axis=1, keepdims=True) * 0.5
    wsum2 = (jnp.sum(w ** 2, axis=1) * 0.5).reshape(1, NUM_CODES)

    idx2d, loss2d, perp2d = _dist_argmin_stats(z_flat, w, zsum2, wsum2)
    min_encoding_indices = idx2d.reshape(NUM_TOKENS)

    zq_flat = _sc_gather(w, min_encoding_indices)

    z_q = jnp.transpose(zq_flat.reshape(z_perm.shape), (0, 3, 1, 2))
    loss = loss2d.reshape(())
    perplexity = perp2d.reshape(())
    return (z_q, min_encoding_indices, loss, perplexity)


# speedup vs baseline: 1.0146x; 1.0146x over previous
import functools

import jax
import jax.numpy as jnp
from jax import lax
from jax.experimental import pallas as pl
from jax.experimental.pallas import tpu as pltpu
from jax.experimental.pallas import tpu_sc as plsc

NUM_CODES = 8192
LATENT_DIM = 256
NUM_TOKENS = 8192
BETA = 0.25


_TOK_TILE = 4096
_CODE_TILE = 2048
_LANES = 128
_HIST_CHUNK = 1024


def _main_body(z_ref, w_ref, zsum2_ref, wsum2_ref,
               idx_ref, loss_ref, perp_ref,
               acc_val, acc_idx, counts, loss_sum):
    i = pl.program_id(0)
    j = pl.program_id(1)
    last_i = pl.num_programs(0) - 1
    last_j = pl.num_programs(1) - 1

    @pl.when(jnp.logical_and(i == 0, j == 0))
    def _():
        counts[...] = jnp.zeros(counts.shape, jnp.float32)
        loss_sum[0, 0] = 0.0

    @pl.when(j == 0)
    def _():
        acc_val[...] = jnp.full(acc_val.shape, jnp.inf, jnp.float32)
        acc_idx[...] = jnp.zeros(acc_idx.shape, jnp.int32)

    dot = lax.dot_general(z_ref[...], w_ref[...],
                          (((1,), (1,)), ((), ())),
                          preferred_element_type=jnp.float32)

    zs2 = zsum2_ref[...]
    av = acc_val[...]
    ai = acc_idx[...]
    for c in range(_CODE_TILE // _LANES):
        sl = slice(c * _LANES, (c + 1) * _LANES)
        d2 = (zs2 + wsum2_ref[:, sl]) - dot[:, sl]
        ioc = (lax.broadcasted_iota(jnp.int32, d2.shape, 1)
               + j * _CODE_TILE + c * _LANES)
        upd = d2 < av
        ai = jnp.where(upd, ioc, ai)
        av = jnp.where(upd, d2, av)

    @pl.when(j != last_j)
    def _():
        acc_val[...] = av
        acc_idx[...] = ai

    @pl.when(j == last_j)
    def _():
        m = jnp.min(av, axis=1, keepdims=True)
        idxt = jnp.min(jnp.where(av == m, ai, jnp.int32(2**30)),
                       axis=1, keepdims=True)
        idx_ref[...] = idxt
        loss_sum[0, 0] += jnp.sum(m) * 2.0
        for c0 in range(0, NUM_CODES, _HIST_CHUNK):
            codes = lax.broadcasted_iota(
                jnp.int32, (1, _HIST_CHUNK), 1) + c0
            eq = idxt == codes
            counts[:, c0:c0 + _HIST_CHUNK] += jnp.sum(
                jnp.where(eq, 1.0, 0.0), axis=0, keepdims=True)

    @pl.when(jnp.logical_and(i == last_i, j == last_j))
    def _():
        p = counts[...] * (1.0 / NUM_TOKENS)
        ent = jnp.sum(p * jnp.log(p + 1e-10))
        perp_ref[0, 0] = jnp.exp(-ent)
        mean = loss_sum[0, 0] * (1.0 / (NUM_TOKENS * LATENT_DIM))
        loss_ref[0, 0] = mean + BETA * mean


def _dist_argmin_stats(z_flat, w, zsum2, wsum2):
    grid = (NUM_TOKENS // _TOK_TILE, NUM_CODES // _CODE_TILE)
    return pl.pallas_call(
        _main_body,
        grid=grid,
        in_specs=[
            pl.BlockSpec((_TOK_TILE, LATENT_DIM), lambda i, j: (i, 0)),
            pl.BlockSpec((_CODE_TILE, LATENT_DIM), lambda i, j: (j, 0)),
            pl.BlockSpec((_TOK_TILE, 1), lambda i, j: (i, 0)),
            pl.BlockSpec((1, _CODE_TILE), lambda i, j: (0, j)),
        ],
        out_specs=[
            pl.BlockSpec((_TOK_TILE, 1), lambda i, j: (i, 0)),
            pl.BlockSpec(memory_space=pltpu.SMEM),
            pl.BlockSpec(memory_space=pltpu.SMEM),
        ],
        out_shape=[
            jax.ShapeDtypeStruct((NUM_TOKENS, 1), jnp.int32),
            jax.ShapeDtypeStruct((1, 1), jnp.float32),
            jax.ShapeDtypeStruct((1, 1), jnp.float32),
        ],
        scratch_shapes=[
            pltpu.VMEM((_TOK_TILE, _LANES), jnp.float32),
            pltpu.VMEM((_TOK_TILE, _LANES), jnp.int32),
            pltpu.VMEM((1, NUM_CODES), jnp.float32),
            pltpu.SMEM((1, 1), jnp.float32),
        ],
        compiler_params=pltpu.CompilerParams(
            dimension_semantics=("arbitrary", "arbitrary")),
    )(z_flat, w, zsum2, wsum2)


_SC_NUM_CORES = 2
_SC_NUM_SUBCORES = 16


def _sc_gather(table, idx):
    nw = _SC_NUM_CORES * _SC_NUM_SUBCORES
    b_per_w = NUM_TOKENS // nw
    mesh = plsc.VectorSubcoreMesh(core_axis_name="c", subcore_axis_name="s",
                                  num_cores=_SC_NUM_CORES,
                                  num_subcores=_SC_NUM_SUBCORES)

    @functools.partial(
        pl.kernel,
        mesh=mesh,
        out_type=jax.ShapeDtypeStruct((NUM_TOKENS, LATENT_DIM), jnp.float32),
        scratch_types=[
            pltpu.VMEM((b_per_w,), jnp.int32),
            pltpu.VMEM((b_per_w, LATENT_DIM), jnp.float32),
            pltpu.SemaphoreType.DMA,
        ],
    )
    def gather_kernel(table_hbm, idx_hbm, out_hbm, idx_v, rows_v, sem):
        wid = lax.axis_index("s") * _SC_NUM_CORES + lax.axis_index("c")
        base = wid * b_per_w
        pltpu.sync_copy(idx_hbm.at[pl.ds(base, b_per_w)], idx_v)
        pltpu.async_copy(table_hbm.at[idx_v], rows_v, sem).wait()
        pltpu.sync_copy(rows_v, out_hbm.at[pl.ds(base, b_per_w)])

    return gather_kernel(table, idx)


def kernel(z, embedding_weight):
    norm = jnp.sqrt(jnp.sum(embedding_weight ** 2, axis=1, keepdims=True))
    w = embedding_weight / jnp.maximum(norm, 1e-12)

    z_perm = jnp.transpose(z, (0, 2, 3, 1))
    z_flat = z_perm.reshape(-1, LATENT_DIM)

    zsum2 = jnp.sum(z_flat ** 2, axis=1, keepdims=True) * 0.5
    wsum2 = (jnp.sum(w ** 2, axis=1) * 0.5).reshape(1, NUM_CODES)

    idx2d, loss2d, perp2d = _dist_argmin_stats(z_flat, w, zsum2, wsum2)
    min_encoding_indices = idx2d.reshape(NUM_TOKENS)

    zq_flat = _sc_gather(w, min_encoding_indices)

    z_q = jnp.transpose(zq_flat.reshape(z_perm.shape), (0, 3, 1, 2))
    loss = loss2d.reshape(())
    perplexity = perp2d.reshape(())
    return (z_q, min_encoding_indices, loss, perplexity)


# speedup vs baseline: 1.0161x; 1.0015x over previous
import functools

import jax
import jax.numpy as jnp
from jax import lax
from jax.experimental import pallas as pl
from jax.experimental.pallas import tpu as pltpu
from jax.experimental.pallas import tpu_sc as plsc

NUM_CODES = 8192
LATENT_DIM = 256
NUM_TOKENS = 8192
BETA = 0.25


_TOK_TILE = 4096
_CODE_TILE = 2048
_LANES = 128


def _main_body(z_ref, w_ref, zsum2_ref, wsum2_ref,
               idx_ref, loss_ref,
               acc_val, acc_idx, loss_sum):
    i = pl.program_id(0)
    j = pl.program_id(1)
    last_i = pl.num_programs(0) - 1
    last_j = pl.num_programs(1) - 1

    @pl.when(jnp.logical_and(i == 0, j == 0))
    def _():
        loss_sum[0, 0] = 0.0

    @pl.when(j == 0)
    def _():
        acc_val[...] = jnp.full(acc_val.shape, jnp.inf, jnp.float32)
        acc_idx[...] = jnp.zeros(acc_idx.shape, jnp.int32)

    dot = lax.dot_general(z_ref[...], w_ref[...],
                          (((1,), (1,)), ((), ())),
                          preferred_element_type=jnp.float32)

    zs2 = zsum2_ref[...]
    av = acc_val[...]
    ai = acc_idx[...]
    for c in range(_CODE_TILE // _LANES):
        sl = slice(c * _LANES, (c + 1) * _LANES)
        d2 = (zs2 + wsum2_ref[:, sl]) - dot[:, sl]
        ioc = (lax.broadcasted_iota(jnp.int32, d2.shape, 1)
               + j * _CODE_TILE + c * _LANES)
        upd = d2 < av
        ai = jnp.where(upd, ioc, ai)
        av = jnp.where(upd, d2, av)

    @pl.when(j != last_j)
    def _():
        acc_val[...] = av
        acc_idx[...] = ai

    @pl.when(j == last_j)
    def _():
        m = jnp.min(av, axis=1, keepdims=True)
        idxt = jnp.min(jnp.where(av == m, ai, jnp.int32(2**30)),
                       axis=1, keepdims=True)
        idx_ref[...] = idxt
        loss_sum[0, 0] += jnp.sum(m) * 2.0

    @pl.when(jnp.logical_and(i == last_i, j == last_j))
    def _():
        mean = loss_sum[0, 0] * (1.0 / (NUM_TOKENS * LATENT_DIM))
        loss_ref[0, 0] = mean + BETA * mean


def _dist_argmin_loss(z_flat, w, zsum2, wsum2):
    grid = (NUM_TOKENS // _TOK_TILE, NUM_CODES // _CODE_TILE)
    return pl.pallas_call(
        _main_body,
        grid=grid,
        in_specs=[
            pl.BlockSpec((_TOK_TILE, LATENT_DIM), lambda i, j: (i, 0)),
            pl.BlockSpec((_CODE_TILE, LATENT_DIM), lambda i, j: (j, 0)),
            pl.BlockSpec((_TOK_TILE, 1), lambda i, j: (i, 0)),
            pl.BlockSpec((1, _CODE_TILE), lambda i, j: (0, j)),
        ],
        out_specs=[
            pl.BlockSpec((_TOK_TILE, 1), lambda i, j: (i, 0)),
            pl.BlockSpec(memory_space=pltpu.SMEM),
        ],
        out_shape=[
            jax.ShapeDtypeStruct((NUM_TOKENS, 1), jnp.int32),
            jax.ShapeDtypeStruct((1, 1), jnp.float32),
        ],
        scratch_shapes=[
            pltpu.VMEM((_TOK_TILE, _LANES), jnp.float32),
            pltpu.VMEM((_TOK_TILE, _LANES), jnp.int32),
            pltpu.SMEM((1, 1), jnp.float32),
        ],
        compiler_params=pltpu.CompilerParams(
            dimension_semantics=("arbitrary", "arbitrary")),
    )(z_flat, w, zsum2, wsum2)


_HIST_CHUNK = 1024


def _hist_body(idx_ref, perp_ref):
    idxv = idx_ref[...]
    ent = jnp.float32(0.0)
    for c0 in range(0, NUM_CODES, _HIST_CHUNK):
        codes = lax.broadcasted_iota(jnp.int32, (1, _HIST_CHUNK), 1) + c0
        eq = idxv == codes
        cnt = jnp.sum(jnp.where(eq, 1.0, 0.0), axis=0, keepdims=True)
        p = cnt * (1.0 / NUM_TOKENS)
        ent += jnp.sum(p * jnp.log(p + 1e-10))
    perp_ref[0, 0] = jnp.exp(-ent)


def _perplexity(idx2d):
    return pl.pallas_call(
        _hist_body,
        in_specs=[pl.BlockSpec((NUM_TOKENS, 1), lambda: (0, 0))],
        out_specs=pl.BlockSpec(memory_space=pltpu.SMEM),
        out_shape=jax.ShapeDtypeStruct((1, 1), jnp.float32),
    )(idx2d)


_SC_NUM_CORES = 2
_SC_NUM_SUBCORES = 16


def _sc_gather(table, idx):
    nw = _SC_NUM_CORES * _SC_NUM_SUBCORES
    b_per_w = NUM_TOKENS // nw
    mesh = plsc.VectorSubcoreMesh(core_axis_name="c", subcore_axis_name="s",
                                  num_cores=_SC_NUM_CORES,
                                  num_subcores=_SC_NUM_SUBCORES)

    @functools.partial(
        pl.kernel,
        mesh=mesh,
        out_type=jax.ShapeDtypeStruct((NUM_TOKENS, LATENT_DIM), jnp.float32),
        scratch_types=[
            pltpu.VMEM((b_per_w,), jnp.int32),
            pltpu.VMEM((b_per_w, LATENT_DIM), jnp.float32),
            pltpu.SemaphoreType.DMA,
        ],
    )
    def gather_kernel(table_hbm, idx_hbm, out_hbm, idx_v, rows_v, sem):
        wid = lax.axis_index("s") * _SC_NUM_CORES + lax.axis_index("c")
        base = wid * b_per_w
        pltpu.sync_copy(idx_hbm.at[pl.ds(base, b_per_w)], idx_v)
        pltpu.async_copy(table_hbm.at[idx_v], rows_v, sem).wait()
        pltpu.sync_copy(rows_v, out_hbm.at[pl.ds(base, b_per_w)])

    return gather_kernel(table, idx)


def kernel(z, embedding_weight):
    norm = jnp.sqrt(jnp.sum(embedding_weight ** 2, axis=1, keepdims=True))
    w = embedding_weight / jnp.maximum(norm, 1e-12)

    z_perm = jnp.transpose(z, (0, 2, 3, 1))
    z_flat = z_perm.reshape(-1, LATENT_DIM)

    zsum2 = jnp.sum(z_flat ** 2, axis=1, keepdims=True) * 0.5
    wsum2 = (jnp.sum(w ** 2, axis=1) * 0.5).reshape(1, NUM_CODES)

    idx2d, loss2d = _dist_argmin_loss(z_flat, w, zsum2, wsum2)
    min_encoding_indices = idx2d.reshape(NUM_TOKENS)

    zq_flat = _sc_gather(w, min_encoding_indices)
    perp2d = _perplexity(idx2d)

    z_q = jnp.transpose(zq_flat.reshape(z_perm.shape), (0, 3, 1, 2))
    loss = loss2d.reshape(())
    perplexity = perp2d.reshape(())
    return (z_q, min_encoding_indices, loss, perplexity)


# speedup vs baseline: 1.0602x; 1.0434x over previous
import functools

import jax
import jax.numpy as jnp
from jax import lax
from jax.experimental import pallas as pl
from jax.experimental.pallas import tpu as pltpu
from jax.experimental.pallas import tpu_sc as plsc

NUM_CODES = 8192
LATENT_DIM = 256
NUM_TOKENS = 8192
BETA = 0.25


_TOK_TILE = 4096
_CODE_TILE = 2048
_LANES = 128


def _main_body(z_ref, w_ref,
               idx_ref, loss_ref,
               acc_val, acc_idx, zsum2_s, wsum2_s, loss_sum):
    i = pl.program_id(0)
    j = pl.program_id(1)
    last_i = pl.num_programs(0) - 1
    last_j = pl.num_programs(1) - 1

    @pl.when(jnp.logical_and(i == 0, j == 0))
    def _():
        loss_sum[0, 0] = 0.0

    @pl.when(j == 0)
    def _():
        acc_val[...] = jnp.full(acc_val.shape, jnp.inf, jnp.float32)
        acc_idx[...] = jnp.zeros(acc_idx.shape, jnp.int32)
        zt = z_ref[...]
        zsum2_s[...] = jnp.sum(zt * zt, axis=1, keepdims=True) * 0.5

    @pl.when(i == 0)
    def _():
        wt = w_ref[...]
        ws = jnp.sum(wt * wt, axis=1, keepdims=True) * 0.5
        wsum2_s[:, pl.ds(j * _CODE_TILE, _CODE_TILE)] = ws.reshape(
            1, _CODE_TILE)

    dot = lax.dot_general(z_ref[...], w_ref[...],
                          (((1,), (1,)), ((), ())),
                          preferred_element_type=jnp.float32)

    zs2 = zsum2_s[...]
    av = acc_val[...]
    ai = acc_idx[...]
    for c in range(_CODE_TILE // _LANES):
        sl = slice(c * _LANES, (c + 1) * _LANES)
        d2 = (zs2 + wsum2_s[:, pl.ds(j * _CODE_TILE + c * _LANES,
                                      _LANES)]) - dot[:, sl]
        ioc = (lax.broadcasted_iota(jnp.int32, d2.shape, 1)
               + j * _CODE_TILE + c * _LANES)
        upd = d2 < av
        ai = jnp.where(upd, ioc, ai)
        av = jnp.where(upd, d2, av)

    @pl.when(j != last_j)
    def _():
        acc_val[...] = av
        acc_idx[...] = ai

    @pl.when(j == last_j)
    def _():
        m = jnp.min(av, axis=1, keepdims=True)
        idxt = jnp.min(jnp.where(av == m, ai, jnp.int32(2**30)),
                       axis=1, keepdims=True)
        idx_ref[...] = idxt
        loss_sum[0, 0] += jnp.sum(m) * 2.0

    @pl.when(jnp.logical_and(i == last_i, j == last_j))
    def _():
        mean = loss_sum[0, 0] * (1.0 / (NUM_TOKENS * LATENT_DIM))
        loss_ref[0, 0] = mean + BETA * mean


def _dist_argmin_loss(z_flat, w):
    grid = (NUM_TOKENS // _TOK_TILE, NUM_CODES // _CODE_TILE)
    return pl.pallas_call(
        _main_body,
        grid=grid,
        in_specs=[
            pl.BlockSpec((_TOK_TILE, LATENT_DIM), lambda i, j: (i, 0)),
            pl.BlockSpec((_CODE_TILE, LATENT_DIM), lambda i, j: (j, 0)),
        ],
        out_specs=[
            pl.BlockSpec((_TOK_TILE, 1), lambda i, j: (i, 0)),
            pl.BlockSpec(memory_space=pltpu.SMEM),
        ],
        out_shape=[
            jax.ShapeDtypeStruct((NUM_TOKENS, 1), jnp.int32),
            jax.ShapeDtypeStruct((1, 1), jnp.float32),
        ],
        scratch_shapes=[
            pltpu.VMEM((_TOK_TILE, _LANES), jnp.float32),
            pltpu.VMEM((_TOK_TILE, _LANES), jnp.int32),
            pltpu.VMEM((_TOK_TILE, 1), jnp.float32),
            pltpu.VMEM((1, NUM_CODES), jnp.float32),
            pltpu.SMEM((1, 1), jnp.float32),
        ],
        compiler_params=pltpu.CompilerParams(
            dimension_semantics=("arbitrary", "arbitrary")),
    )(z_flat, w)


_HIST_CHUNK = 1024


def _hist_body(idx_ref, perp_ref):
    idxv = idx_ref[...]
    ent = jnp.float32(0.0)
    for c0 in range(0, NUM_CODES, _HIST_CHUNK):
        codes = lax.broadcasted_iota(jnp.int32, (1, _HIST_CHUNK), 1) + c0
        eq = idxv == codes
        cnt = jnp.sum(jnp.where(eq, 1.0, 0.0), axis=0, keepdims=True)
        p = cnt * (1.0 / NUM_TOKENS)
        ent += jnp.sum(p * jnp.log(p + 1e-10))
    perp_ref[0, 0] = jnp.exp(-ent)


def _perplexity(idx2d):
    return pl.pallas_call(
        _hist_body,
        in_specs=[pl.BlockSpec((NUM_TOKENS, 1), lambda: (0, 0))],
        out_specs=pl.BlockSpec(memory_space=pltpu.SMEM),
        out_shape=jax.ShapeDtypeStruct((1, 1), jnp.float32),
    )(idx2d)


_SC_NUM_CORES = 2
_SC_NUM_SUBCORES = 16


def _sc_gather(table, idx):
    nw = _SC_NUM_CORES * _SC_NUM_SUBCORES
    b_per_w = NUM_TOKENS // nw
    mesh = plsc.VectorSubcoreMesh(core_axis_name="c", subcore_axis_name="s",
                                  num_cores=_SC_NUM_CORES,
                                  num_subcores=_SC_NUM_SUBCORES)

    @functools.partial(
        pl.kernel,
        mesh=mesh,
        out_type=jax.ShapeDtypeStruct((NUM_TOKENS, LATENT_DIM), jnp.float32),
        scratch_types=[
            pltpu.VMEM((b_per_w,), jnp.int32),
            pltpu.VMEM((b_per_w, LATENT_DIM), jnp.float32),
            pltpu.SemaphoreType.DMA,
        ],
    )
    def gather_kernel(table_hbm, idx_hbm, out_hbm, idx_v, rows_v, sem):
        wid = lax.axis_index("s") * _SC_NUM_CORES + lax.axis_index("c")
        base = wid * b_per_w
        pltpu.sync_copy(idx_hbm.at[pl.ds(base, b_per_w)], idx_v)
        pltpu.async_copy(table_hbm.at[idx_v], rows_v, sem).wait()
        pltpu.sync_copy(rows_v, out_hbm.at[pl.ds(base, b_per_w)])

    return gather_kernel(table, idx)


def kernel(z, embedding_weight):
    norm = jnp.sqrt(jnp.sum(embedding_weight ** 2, axis=1, keepdims=True))
    w = embedding_weight / jnp.maximum(norm, 1e-12)

    z_perm = jnp.transpose(z, (0, 2, 3, 1))
    z_flat = z_perm.reshape(-1, LATENT_DIM)

    idx2d, loss2d = _dist_argmin_loss(z_flat, w)
    min_encoding_indices = idx2d.reshape(NUM_TOKENS)

    zq_flat = _sc_gather(w, min_encoding_indices)
    perp2d = _perplexity(idx2d)

    z_q = jnp.transpose(zq_flat.reshape(z_perm.shape), (0, 3, 1, 2))
    loss = loss2d.reshape(())
    perplexity = perp2d.reshape(())
    return (z_q, min_encoding_indices, loss, perplexity)
